# baseline (speedup 1.0000x reference)
import jax
import jax.numpy as jnp
from jax import lax
from jax.experimental import pallas as pl
from jax.experimental.pallas import tpu as pltpu
from jax.experimental.pallas import tpu_sc as plsc

B = 16384
D = 64
NC = 2
NS = 16
NW = NC * NS
BPW = B // NW
GCH = 128
NGCH = BPW // GCH
CH = 16
NCH = BPW // CH
TPR = 8

_SCR = 17 * 16

_MESH = plsc.VectorSubcoreMesh(
    core_axis_name="c", subcore_axis_name="s",
    num_cores=NC, num_subcores=NS)


def _wid_base():
  wid = lax.axis_index("s") * NC + lax.axis_index("c")
  return wid * BPW


def _body_a(item_p_hbm, item_n_hbm, ifl, pn_out,
            idx_p, idx_n, p_rows, n_rows, sem_p, sem_n):
  base = _wid_base()
  for j in range(NGCH):
    pltpu.sync_copy(item_p_hbm.at[pl.ds(base + j * GCH, GCH)], idx_p.at[j])
    pltpu.sync_copy(item_n_hbm.at[pl.ds(base + j * GCH, GCH)], idx_n.at[j])
  copies = []
  for j in range(NGCH):
    sl = pl.ds(j * GCH, GCH)
    copies.append(pltpu.async_copy(ifl.at[idx_p.at[j]], p_rows.at[sl], sem_p))
    copies.append(pltpu.async_copy(ifl.at[idx_n.at[j]], n_rows.at[sl], sem_n))
  for c in copies:
    c.wait()

  def sub(g, carry):
    r0 = g * 16
    for j in range(16):
      for q in range(D // 16):
        sl = pl.ds(q * 16, 16)
        p_rows[r0 + j, sl] = p_rows[r0 + j, sl] - n_rows[r0 + j, sl]
    return carry

  lax.fori_loop(0, BPW // 16, sub, 0)
  pltpu.sync_copy(p_rows, pn_out.at[pl.ds(base, BPW)])


def _body_b1(user_hbm, uf3, u_out, idx_u, t0, t1, u_flat, sem0, sem1):
  base = _wid_base()
  pltpu.sync_copy(user_hbm.at[pl.ds(base, BPW)], idx_u)
  bufs = (t0, t1)
  sems = (sem0, sem1)

  def load_idx(c):
    return idx_u[pl.ds(c * CH, CH)]

  def fire(iv, slot):
    tv = iv >> 3
    for j in range(CH):
      pltpu.async_copy(uf3.at[tv[j]], bufs[slot].at[j], sems[slot])

  def wait(slot):
    for j in range(CH):
      pltpu.make_async_copy(uf3.at[0], bufs[slot].at[j], sems[slot]).wait()

  def emit(c, iv, slot):
    tb = bufs[slot]
    for j in range(CH):
      qu = iv[j] & (TPR - 1)
      for q in range(D // 16):
        sl = pl.ds(q * 16, 16)
        u_flat[pl.ds(c * CH * D + j * D + q * 16, 16)] = tb[j, qu, sl]

  fire(load_idx(0), 0)

  def pair(pi, carry):
    c0 = 2 * pi
    cur0 = carry
    cur1 = load_idx(c0 + 1)
    fire(cur1, 1)
    wait(0)
    emit(c0, cur0, 0)
    nxt0 = load_idx(jnp.minimum(c0 + 2, NCH - 2))

    @pl.when(pi < NCH // 2 - 1)
    def _():
      fire(nxt0, 0)

    wait(1)
    emit(c0 + 1, cur1, 1)
    return nxt0

  lax.fori_loop(0, NCH // 2, pair, load_idx(0))
  pltpu.sync_copy(u_flat, u_out.at[pl.ds(base * D, BPW * D)])


def _body_b2(u_hbm, pn_hbm, out_hbm, u_v, pn_v, out_v, scr, sem_u, sem_p):
  base = _wid_base()
  cu = pltpu.make_async_copy(u_hbm.at[pl.ds(base * D, BPW * D)], u_v, sem_u)
  cp = pltpu.make_async_copy(pn_hbm.at[pl.ds(base, BPW)], pn_v, sem_p)
  cu.start()
  cp.start()
  cu.wait()
  cp.wait()

  lane = lax.iota(jnp.int32, 16)
  lane17 = lane * 17

  def grp(g, carry):
    g0 = g * 16
    for j in range(16):
      r = g0 + j
      acc = None
      for q in range(D // 16):
        sl = pl.ds(q * 16, 16)
        t = u_v[pl.ds(r * D + q * 16, 16)] * pn_v[r, sl]
        acc = t if acc is None else acc + t
      plsc.store_scatter(scr, [lane17 + j], acc)
    tot = None
    for d in range(16):
      v = scr[pl.ds(d * 17, 16)]
      tot = v if tot is None else tot + v
    out_v[pl.ds(g * 16, 16)] = tot
    return carry

  lax.fori_loop(0, BPW // 16, grp, 0)
  pltpu.sync_copy(out_v, out_hbm.at[pl.ds(base, BPW)])


@jax.jit
def kernel(user, item_p, item_n, user_factors, item_factors):
  n_items = item_factors.shape[0]
  if_lin = lax.optimization_barrier(item_factors.reshape(-1)).reshape(
      n_items, D)

  a = pl.kernel(
      _body_a,
      out_type=jax.ShapeDtypeStruct((B, D), jnp.float32),
      mesh=_MESH,
      compiler_params=pltpu.CompilerParams(
          needs_layout_passes=False, use_tc_tiling_on_sc=False),
      scratch_types=[
          pltpu.VMEM((NGCH, GCH), jnp.int32),
          pltpu.VMEM((NGCH, GCH), jnp.int32),
          pltpu.VMEM((BPW, D), jnp.float32),
          pltpu.VMEM((BPW, D), jnp.float32),
          pltpu.SemaphoreType.DMA,
          pltpu.SemaphoreType.DMA,
      ],
  )
  pn = a(item_p, item_n, if_lin)

  uf3 = user_factors.reshape(user_factors.shape[0] // TPR, TPR, D)
  tile = pltpu.VMEM((CH, TPR, D), jnp.float32)
  b1 = pl.kernel(
      _body_b1,
      out_type=jax.ShapeDtypeStruct((B * D,), jnp.float32),
      mesh=_MESH,
      compiler_params=pltpu.CompilerParams(needs_layout_passes=False),
      scratch_types=[
          pltpu.VMEM((BPW,), jnp.int32),
          tile, tile,
          pltpu.VMEM((BPW * D,), jnp.float32),
          pltpu.SemaphoreType.DMA,
          pltpu.SemaphoreType.DMA,
      ],
  )
  u_rows = b1(user, uf3)

  b2 = pl.kernel(
      _body_b2,
      out_type=jax.ShapeDtypeStruct((B,), jnp.float32),
      mesh=_MESH,
      compiler_params=pltpu.CompilerParams(
          needs_layout_passes=False, use_tc_tiling_on_sc=False),
      scratch_types=[
          pltpu.VMEM((BPW * D,), jnp.float32),
          pltpu.VMEM((BPW, D), jnp.float32),
          pltpu.VMEM((BPW,), jnp.float32),
          pltpu.VMEM((_SCR,), jnp.float32),
          pltpu.SemaphoreType.DMA,
          pltpu.SemaphoreType.DMA,
      ],
  )
  return b2(u_rows, pn)


# speedup vs baseline: 1.0209x; 1.0209x over previous
import jax
import jax.numpy as jnp
from jax import lax
from jax.experimental import pallas as pl
from jax.experimental.pallas import tpu as pltpu
from jax.experimental.pallas import tpu_sc as plsc

B = 16384
D = 64
NC = 2
NS = 16
NW = NC * NS
BPW = B // NW
CH = 16
NCH = BPW // CH
TPR = 8

_SCR = 17 * 16


def _body(user_hbm, item_p_hbm, item_n_hbm, uf3, if3, out_hbm,
          idx_u, idx_p, idx_n,
          u_t0, p_t0, n_t0, u_t1, p_t1, n_t1,
          out_v, scr,
          sem_u0, sem_p0, sem_n0, sem_u1, sem_p1, sem_n1):
  wid = lax.axis_index("s") * NC + lax.axis_index("c")
  base = wid * BPW

  pltpu.sync_copy(user_hbm.at[pl.ds(base, BPW)], idx_u)
  pltpu.sync_copy(item_p_hbm.at[pl.ds(base, BPW)], idx_p)
  pltpu.sync_copy(item_n_hbm.at[pl.ds(base, BPW)], idx_n)

  lane = lax.iota(jnp.int32, 16)
  lane17 = lane * 17

  bufs = ((u_t0, p_t0, n_t0), (u_t1, p_t1, n_t1))
  sems = ((sem_u0, sem_p0, sem_n0), (sem_u1, sem_p1, sem_n1))

  def load_idx(c):
    c0 = c * CH
    return (idx_u[pl.ds(c0, CH)], idx_p[pl.ds(c0, CH)], idx_n[pl.ds(c0, CH)])

  def fire(idxs, slot):
    for (src, buf, sem), iv in zip(
        ((uf3, bufs[slot][0], sems[slot][0]),
         (if3, bufs[slot][1], sems[slot][1]),
         (if3, bufs[slot][2], sems[slot][2])), idxs):
      tv = (iv >> 3) << 3
      for j in range(CH):
        pltpu.async_copy(
            src.at[pl.ds(pl.multiple_of(tv[j], TPR), TPR)], buf.at[j], sem)

  def wait(slot):
    for src, buf, sem in (
        (uf3, bufs[slot][0], sems[slot][0]),
        (if3, bufs[slot][1], sems[slot][1]),
        (if3, bufs[slot][2], sems[slot][2])):
      for j in range(CH):
        pltpu.make_async_copy(src.at[pl.ds(0, TPR)], buf.at[j], sem).wait()

  def compute(c, idxs, slot):
    ub, pb, nb = bufs[slot]
    iu, ip, inn = idxs
    for j in range(CH):
      qu = iu[j] & (TPR - 1)
      qp = ip[j] & (TPR - 1)
      qn = inn[j] & (TPR - 1)
      acc = None
      for q in range(D // 16):
        sl = pl.ds(q * 16, 16)
        t = ub[j, qu, sl] * (pb[j, qp, sl] - nb[j, qn, sl])
        acc = t if acc is None else acc + t
      plsc.store_scatter(scr, [lane17 + j], acc)
    tot = None
    for d in range(16):
      v = scr[pl.ds(d * 17, 16)]
      tot = v if tot is None else tot + v
    out_v[pl.ds(c * CH, 16)] = tot

  fire(load_idx(0), 0)

  def pair(pi, carry):
    c0 = 2 * pi
    cur0 = carry
    cur1 = load_idx(c0 + 1)
    fire(cur1, 1)
    wait(0)
    compute(c0, cur0, 0)
    nxt0 = load_idx(jnp.minimum(c0 + 2, NCH - 2))

    @pl.when(pi < NCH // 2 - 1)
    def _():
      fire(nxt0, 0)

    wait(1)
    compute(c0 + 1, cur1, 1)
    return nxt0

  lax.fori_loop(0, NCH // 2, pair, load_idx(0))
  pltpu.sync_copy(out_v, out_hbm.at[pl.ds(base, BPW)])


@jax.jit
def kernel(user, item_p, item_n, user_factors, item_factors):
  uf3 = user_factors
  if3 = item_factors
  mesh = plsc.VectorSubcoreMesh(
      core_axis_name="c", subcore_axis_name="s",
      num_cores=NC, num_subcores=NS)
  tile = pltpu.VMEM((CH, TPR, D), jnp.float32)
  k = pl.kernel(
      _body,
      out_type=jax.ShapeDtypeStruct((B,), jnp.float32),
      mesh=mesh,
      compiler_params=pltpu.CompilerParams(needs_layout_passes=False),
      scratch_types=[
          pltpu.VMEM((BPW,), jnp.int32),
          pltpu.VMEM((BPW,), jnp.int32),
          pltpu.VMEM((BPW,), jnp.int32),
          tile, tile, tile, tile, tile, tile,
          pltpu.VMEM((BPW,), jnp.float32),
          pltpu.VMEM((_SCR,), jnp.float32),
          pltpu.SemaphoreType.DMA,
          pltpu.SemaphoreType.DMA,
          pltpu.SemaphoreType.DMA,
          pltpu.SemaphoreType.DMA,
          pltpu.SemaphoreType.DMA,
          pltpu.SemaphoreType.DMA,
      ],
  )
  return k(user, item_p, item_n, uf3, if3)


# speedup vs baseline: 1.0212x; 1.0004x over previous
import jax
import jax.numpy as jnp
from jax import lax
from jax.experimental import pallas as pl
from jax.experimental.pallas import tpu as pltpu
from jax.experimental.pallas import tpu_sc as plsc

B = 16384
D = 64
NC = 2
NS = 16
NW = NC * NS
BPW = B // NW
CH = 16
NCH = BPW // CH
TPR = 8

_SCR = 17 * 16


def _body(user_hbm, item_p_hbm, item_n_hbm, uf_hbm, if_hbm, out_hbm,
          idx_u, idx_p, idx_n,
          u_t0, p_t0, n_t0, u_t1, p_t1, n_t1,
          out_v, scr,
          sem_u0, sem_p0, sem_n0, sem_u1, sem_p1, sem_n1):
  wid = lax.axis_index("s") * NC + lax.axis_index("c")
  base = wid * BPW

  pltpu.sync_copy(user_hbm.at[pl.ds(base, BPW)], idx_u)
  pltpu.sync_copy(item_p_hbm.at[pl.ds(base, BPW)], idx_p)
  pltpu.sync_copy(item_n_hbm.at[pl.ds(base, BPW)], idx_n)

  lane = lax.iota(jnp.int32, 16)
  lane17 = lane * 17

  uf3 = uf_hbm.reshape(uf_hbm.shape[0] // TPR, TPR, D)
  if3 = if_hbm.reshape(if_hbm.shape[0] // TPR, TPR, D)

  bufs = ((u_t0, p_t0, n_t0), (u_t1, p_t1, n_t1))
  sems = ((sem_u0, sem_p0, sem_n0), (sem_u1, sem_p1, sem_n1))

  def load_idx(c):
    c0 = c * CH
    return (idx_u[pl.ds(c0, CH)], idx_p[pl.ds(c0, CH)], idx_n[pl.ds(c0, CH)])

  def fire(idxs, slot):
    for (src, buf, sem), iv in zip(
        ((uf3, bufs[slot][0], sems[slot][0]),
         (if3, bufs[slot][1], sems[slot][1]),
         (if3, bufs[slot][2], sems[slot][2])), idxs):
      tv = iv >> 3
      for j in range(CH):
        pltpu.async_copy(src.at[tv[j]], buf.at[j], sem)

  def wait(slot):
    for src, buf, sem in (
        (uf3, bufs[slot][0], sems[slot][0]),
        (if3, bufs[slot][1], sems[slot][1]),
        (if3, bufs[slot][2], sems[slot][2])):
      for j in range(CH):
        pltpu.make_async_copy(src.at[0], buf.at[j], sem).wait()

  def compute(c, idxs, slot):
    ub, pb, nb = bufs[slot]
    iu, ip, inn = idxs
    for j in range(CH):
      qu = iu[j] & (TPR - 1)
      qp = ip[j] & (TPR - 1)
      qn = inn[j] & (TPR - 1)
      acc = None
      for q in range(D // 16):
        sl = pl.ds(q * 16, 16)
        t = ub[j, qu, sl] * (pb[j, qp, sl] - nb[j, qn, sl])
        acc = t if acc is None else acc + t
      plsc.store_scatter(scr, [lane17 + j], acc)
    tot = None
    for d in range(16):
      v = scr[pl.ds(d * 17, 16)]
      tot = v if tot is None else tot + v
    out_v[pl.ds(c * CH, 16)] = tot

  fire(load_idx(0), 0)

  def pair(pi, carry):
    c0 = 2 * pi
    cur0 = carry
    cur1 = load_idx(c0 + 1)
    fire(cur1, 1)
    wait(0)
    compute(c0, cur0, 0)
    nxt0 = load_idx(jnp.minimum(c0 + 2, NCH - 2))

    @pl.when(pi < NCH // 2 - 1)
    def _():
      fire(nxt0, 0)

    wait(1)
    compute(c0 + 1, cur1, 1)
    return nxt0

  lax.fori_loop(0, NCH // 2, pair, load_idx(0))
  pltpu.sync_copy(out_v, out_hbm.at[pl.ds(base, BPW)])


@jax.jit
def kernel(user, item_p, item_n, user_factors, item_factors):
  mesh = plsc.VectorSubcoreMesh(
      core_axis_name="c", subcore_axis_name="s",
      num_cores=NC, num_subcores=NS)
  tile = pltpu.VMEM((CH, TPR, D), jnp.float32)
  k = pl.kernel(
      _body,
      out_type=jax.ShapeDtypeStruct((B,), jnp.float32),
      mesh=mesh,
      compiler_params=pltpu.CompilerParams(needs_layout_passes=False),
      scratch_types=[
          pltpu.VMEM((BPW,), jnp.int32),
          pltpu.VMEM((BPW,), jnp.int32),
          pltpu.VMEM((BPW,), jnp.int32),
          tile, tile, tile, tile, tile, tile,
          pltpu.VMEM((BPW,), jnp.float32),
          pltpu.VMEM((_SCR,), jnp.float32),
          pltpu.SemaphoreType.DMA,
          pltpu.SemaphoreType.DMA,
          pltpu.SemaphoreType.DMA,
          pltpu.SemaphoreType.DMA,
          pltpu.SemaphoreType.DMA,
          pltpu.SemaphoreType.DMA,
      ],
  )
  return k(user, item_p, item_n, user_factors, item_factors)
